# baseline (speedup 1.0000x reference)
import jax
import jax.numpy as jnp
from jax.experimental import pallas as pl
from jax.experimental.pallas import tpu as pltpu

_K = 5
_D = 512
_N = 8192
_BQ = 256
_NEG = float("-inf")
_BIGI = 2**30


def _norm_body(x_ref, o_ref):
    x = x_ref[...]
    n = jnp.sqrt(jnp.sum(x * x, axis=-1, keepdims=True))
    o_ref[...] = x / jnp.maximum(n, 1e-12)


def _topk_body(q_ref, e_ref, s_ref, i_ref):
    q = q_ref[...]
    qn = q / jnp.maximum(
        jnp.sqrt(jnp.sum(q * q, axis=-1, keepdims=True)), 1e-12)
    vals = jax.lax.dot_general(
        qn, e_ref[...], (((1,), (1,)), ((), ())),
        preferred_element_type=jnp.float32)

    ms = [jnp.max(vals, axis=1, keepdims=True)]
    for _ in range(_K - 1):
        ms.append(jnp.max(jnp.where(vals < ms[-1], vals, _NEG),
                          axis=1, keepdims=True))

    k = jax.lax.broadcasted_iota(jnp.int32, (_N, 8), 0)
    col = jax.lax.broadcasted_iota(jnp.int32, (_N, 8), 1)
    rhs = jnp.where(col == 0, k >> 6,
                    jnp.where(col == 1, k & 63, 0)).astype(jnp.float32)
    ii = []
    for m in ms:
        mask = (vals == m).astype(jnp.float32)
        r = jax.lax.dot_general(
            mask, rhs, (((1,), (0,)), ((), ())),
            preferred_element_type=jnp.float32)
        ii.append(r[:, 0:1] * 64.0 + r[:, 1:2])

    s_ref[...] = jnp.concatenate(ms, axis=1)
    i_ref[...] = jnp.concatenate(ii, axis=1).astype(jnp.int32)


def kernel(query, embeddings, top_k):
    del top_k
    b, s, d = query.shape
    q2 = query.reshape(b * s, d)

    en = pl.pallas_call(
        _norm_body,
        grid=(4,),
        in_specs=[pl.BlockSpec((_N // 4, _D), lambda j: (j, 0))],
        out_specs=pl.BlockSpec((_N // 4, _D), lambda j: (j, 0)),
        out_shape=jax.ShapeDtypeStruct((_N, _D), jnp.float32),
    )(embeddings)

    nq = b * s
    scores, indices = pl.pallas_call(
        _topk_body,
        grid=(nq // _BQ,),
        in_specs=[
            pl.BlockSpec((_BQ, _D), lambda i: (i, 0)),
            pl.BlockSpec((_N, _D), lambda i: (0, 0)),
        ],
        out_specs=[
            pl.BlockSpec((_BQ, _K), lambda i: (i, 0)),
            pl.BlockSpec((_BQ, _K), lambda i: (i, 0)),
        ],
        out_shape=[
            jax.ShapeDtypeStruct((nq, _K), jnp.float32),
            jax.ShapeDtypeStruct((nq, _K), jnp.int32),
        ],
        compiler_params=pltpu.CompilerParams(
            dimension_semantics=("arbitrary",)),
    )(q2, en)

    return scores.reshape(b, s, _K), indices.reshape(b, s, _K)


# speedup vs baseline: 1.1313x; 1.1313x over previous
import jax
import jax.numpy as jnp
from jax.experimental import pallas as pl
from jax.experimental.pallas import tpu as pltpu

_K = 5
_D = 512
_N = 8192
_BQ = 256
_NEG = float("-inf")
_BIGI = 2**30


def _norm_body(x_ref, o_ref):
    x = x_ref[...]
    n = jnp.sqrt(jnp.sum(x * x, axis=-1, keepdims=True))
    o_ref[...] = x / jnp.maximum(n, 1e-12)


def _topk_body(q_ref, e_ref, s_ref, i_ref):
    q = q_ref[...]
    qn = q / jnp.maximum(
        jnp.sqrt(jnp.sum(q * q, axis=-1, keepdims=True)), 1e-12)
    vals = jax.lax.dot_general(
        qn, e_ref[...], (((1,), (1,)), ((), ())),
        preferred_element_type=jnp.float32)

    iota = jax.lax.broadcasted_iota(jnp.int32, vals.shape, 1)

    ss, ii = [], []
    for _ in range(_K):
        m = jnp.max(vals, axis=1, keepdims=True)
        sel = jnp.min(jnp.where(vals == m, iota, _BIGI),
                      axis=1, keepdims=True)
        ss.append(m)
        ii.append(sel)
        vals = jnp.where(iota == sel, _NEG, vals)

    s_ref[...] = jnp.concatenate(ss, axis=1)
    i_ref[...] = jnp.concatenate(ii, axis=1)


def kernel(query, embeddings, top_k):
    del top_k
    b, s, d = query.shape
    q2 = query.reshape(b * s, d)

    en = pl.pallas_call(
        _norm_body,
        grid=(4,),
        in_specs=[pl.BlockSpec((_N // 4, _D), lambda j: (j, 0))],
        out_specs=pl.BlockSpec((_N // 4, _D), lambda j: (j, 0)),
        out_shape=jax.ShapeDtypeStruct((_N, _D), jnp.float32),
    )(embeddings)

    nq = b * s
    scores, indices = pl.pallas_call(
        _topk_body,
        grid=(nq // _BQ,),
        in_specs=[
            pl.BlockSpec((_BQ, _D), lambda i: (i, 0)),
            pl.BlockSpec((_N, _D), lambda i: (0, 0)),
        ],
        out_specs=[
            pl.BlockSpec((_BQ, _K), lambda i: (i, 0)),
            pl.BlockSpec((_BQ, _K), lambda i: (i, 0)),
        ],
        out_shape=[
            jax.ShapeDtypeStruct((nq, _K), jnp.float32),
            jax.ShapeDtypeStruct((nq, _K), jnp.int32),
        ],
        compiler_params=pltpu.CompilerParams(
            dimension_semantics=("arbitrary",)),
    )(q2, en)

    return scores.reshape(b, s, _K), indices.reshape(b, s, _K)


# speedup vs baseline: 1.1979x; 1.0589x over previous
import jax
import jax.numpy as jnp
from jax.experimental import pallas as pl
from jax.experimental.pallas import tpu as pltpu

_K = 5
_D = 512
_N = 8192
_BQ = 256
_NEG = float("-inf")
_BIGI = 2**30


def _norm_body(x_ref, o_ref):
    x = x_ref[...]
    n = jnp.sqrt(jnp.sum(x * x, axis=-1, keepdims=True))
    o_ref[...] = x / jnp.maximum(n, 1e-12)


def _topk_body(q_ref, e_ref, s_ref, i_ref):
    q = q_ref[...]
    qn = q / jnp.maximum(
        jnp.sqrt(jnp.sum(q * q, axis=-1, keepdims=True)), 1e-12)
    vals = jax.lax.dot_general(
        qn, e_ref[...], (((1,), (1,)), ((), ())),
        preferred_element_type=jnp.float32)

    iota = jax.lax.broadcasted_iota(jnp.int32, vals.shape, 1)

    ss, ii = [], []
    for _ in range(_K):
        m = jnp.max(vals, axis=1, keepdims=True)
        hit = vals == m
        sel = jnp.min(jnp.where(hit, iota, _BIGI), axis=1, keepdims=True)
        ss.append(m)
        ii.append(sel)
        vals = jnp.where(hit, _NEG, vals)

    s_ref[...] = jnp.concatenate(ss, axis=1)
    i_ref[...] = jnp.concatenate(ii, axis=1)


def kernel(query, embeddings, top_k):
    del top_k
    b, s, d = query.shape
    q2 = query.reshape(b * s, d)

    en = pl.pallas_call(
        _norm_body,
        grid=(4,),
        in_specs=[pl.BlockSpec((_N // 4, _D), lambda j: (j, 0))],
        out_specs=pl.BlockSpec((_N // 4, _D), lambda j: (j, 0)),
        out_shape=jax.ShapeDtypeStruct((_N, _D), jnp.float32),
    )(embeddings)

    nq = b * s
    scores, indices = pl.pallas_call(
        _topk_body,
        grid=(nq // _BQ,),
        in_specs=[
            pl.BlockSpec((_BQ, _D), lambda i: (i, 0)),
            pl.BlockSpec((_N, _D), lambda i: (0, 0)),
        ],
        out_specs=[
            pl.BlockSpec((_BQ, _K), lambda i: (i, 0)),
            pl.BlockSpec((_BQ, _K), lambda i: (i, 0)),
        ],
        out_shape=[
            jax.ShapeDtypeStruct((nq, _K), jnp.float32),
            jax.ShapeDtypeStruct((nq, _K), jnp.int32),
        ],
        compiler_params=pltpu.CompilerParams(
            dimension_semantics=("arbitrary",)),
    )(q2, en)

    return scores.reshape(b, s, _K), indices.reshape(b, s, _K)


# speedup vs baseline: 1.2002x; 1.0020x over previous
import jax
import jax.numpy as jnp
from jax.experimental import pallas as pl
from jax.experimental.pallas import tpu as pltpu

_K = 5
_D = 512
_N = 8192
_BQ = 256
_NEG = float("-inf")
_BIGI = 2**30


def _norm_body(x_ref, o_ref):
    x = x_ref[...]
    n = jnp.sqrt(jnp.sum(x * x, axis=-1, keepdims=True))
    o_ref[...] = x / jnp.maximum(n, 1e-12)


def _topk_body(q_ref, e_ref, s_ref, i_ref):
    q = q_ref[...]
    qn = q / jnp.maximum(
        jnp.sqrt(jnp.sum(q * q, axis=-1, keepdims=True)), 1e-12)
    vals = jax.lax.dot_general(
        qn, e_ref[...], (((1,), (1,)), ((), ())),
        preferred_element_type=jnp.float32)

    iota = jax.lax.broadcasted_iota(jnp.int32, vals.shape, 1)

    ss, ii = [], []
    for _ in range(_K):
        m = jnp.max(vals, axis=1, keepdims=True)
        hit = vals == m
        sel = jnp.min(jnp.where(hit, iota, _BIGI), axis=1, keepdims=True)
        ss.append(m)
        ii.append(sel)
        vals = jnp.where(hit, _NEG, vals)

    s_ref[...] = jnp.concatenate(ss, axis=1)
    i_ref[...] = jnp.concatenate(ii, axis=1)


def kernel(query, embeddings, top_k):
    del top_k
    b, s, d = query.shape
    q2 = query.reshape(b * s, d)

    en = pl.pallas_call(
        _norm_body,
        grid=(4,),
        in_specs=[pl.BlockSpec((_N // 4, _D), lambda j: (j, 0))],
        out_specs=pl.BlockSpec((_N // 4, _D), lambda j: (j, 0)),
        out_shape=jax.ShapeDtypeStruct((_N, _D), jnp.float32),
    )(embeddings)

    nq = b * s
    scores, indices = pl.pallas_call(
        _topk_body,
        grid=(nq // _BQ,),
        in_specs=[
            pl.BlockSpec((_BQ, _D), lambda i: (i, 0)),
            pl.BlockSpec((_N, _D), lambda i: (0, 0)),
        ],
        out_specs=[
            pl.BlockSpec((_BQ, _K), lambda i: (i, 0)),
            pl.BlockSpec((_BQ, _K), lambda i: (i, 0)),
        ],
        out_shape=[
            jax.ShapeDtypeStruct((nq, _K), jnp.float32),
            jax.ShapeDtypeStruct((nq, _K), jnp.int32),
        ],
        compiler_params=pltpu.CompilerParams(
            dimension_semantics=("parallel",)),
    )(q2, en)

    return scores.reshape(b, s, _K), indices.reshape(b, s, _K)


# speedup vs baseline: 1.2006x; 1.0003x over previous
import jax
import jax.numpy as jnp
from jax.experimental import pallas as pl
from jax.experimental.pallas import tpu as pltpu

_K = 5
_D = 512
_N = 8192
_BQ = 256
_NEG = float("-inf")
_BIGI = 2**30


def _norm_body(x_ref, o_ref):
    x = x_ref[...]
    n = jnp.sqrt(jnp.sum(x * x, axis=-1, keepdims=True))
    o_ref[...] = x / jnp.maximum(n, 1e-12)


def _topk_body(q_ref, e_ref, s_ref, i_ref):
    q = q_ref[...]
    qn = q / jnp.maximum(
        jnp.sqrt(jnp.sum(q * q, axis=-1, keepdims=True)), 1e-12)
    vals = jax.lax.dot_general(
        qn, e_ref[...], (((1,), (1,)), ((), ())),
        preferred_element_type=jnp.float32)

    iota = jax.lax.broadcasted_iota(jnp.int32, vals.shape, 1)

    ss, ii = [], []
    for t in range(_K):
        m = jnp.max(vals, axis=1, keepdims=True)
        hit = vals == m
        sel = jnp.min(jnp.where(hit, iota, _BIGI), axis=1, keepdims=True)
        ss.append(m)
        ii.append(sel)
        if t + 1 < _K:
            vals = jnp.where(hit, _NEG, vals)

    s_ref[...] = jnp.concatenate(ss, axis=1)
    i_ref[...] = jnp.concatenate(ii, axis=1)


def kernel(query, embeddings, top_k):
    del top_k
    b, s, d = query.shape
    q2 = query.reshape(b * s, d)

    en = pl.pallas_call(
        _norm_body,
        grid=(4,),
        in_specs=[pl.BlockSpec((_N // 4, _D), lambda j: (j, 0))],
        out_specs=pl.BlockSpec((_N // 4, _D), lambda j: (j, 0)),
        out_shape=jax.ShapeDtypeStruct((_N, _D), jnp.float32),
    )(embeddings)

    nq = b * s
    scores, indices = pl.pallas_call(
        _topk_body,
        grid=(nq // _BQ,),
        in_specs=[
            pl.BlockSpec((_BQ, _D), lambda i: (i, 0)),
            pl.BlockSpec((_N, _D), lambda i: (0, 0)),
        ],
        out_specs=[
            pl.BlockSpec((_BQ, _K), lambda i: (i, 0)),
            pl.BlockSpec((_BQ, _K), lambda i: (i, 0)),
        ],
        out_shape=[
            jax.ShapeDtypeStruct((nq, _K), jnp.float32),
            jax.ShapeDtypeStruct((nq, _K), jnp.int32),
        ],
        compiler_params=pltpu.CompilerParams(
            dimension_semantics=("parallel",)),
    )(q2, en)

    return scores.reshape(b, s, _K), indices.reshape(b, s, _K)
